# Initial kernel scaffold; baseline (speedup 1.0000x reference)
import functools

import jax
import jax.numpy as jnp
from jax import lax
from jax.experimental import pallas as pl
from jax.experimental.pallas import tpu as pltpu
from jax.experimental.pallas import tpu_sc as plsc

_B, _T, _D, _CD, _K = 16, 576, 768, 64, 8192
_N = _B * _T

_M = 512
_KT = 2048


def _k1_body(x_ref, win_ref, bin_ref, cbt_ref, idx_ref, d2_ref):
    lat = jnp.dot(x_ref[...], win_ref[...],
                  preferred_element_type=jnp.float32) + bin_ref[...]
    l2 = jnp.sum(lat * lat, axis=1, keepdims=True)
    lat2 = lat * 2.0

    def tile(k, carry):
        m, am = carry
        cbt = cbt_ref[:, pl.ds(k * _KT, _KT)]
        s = jnp.dot(lat2, cbt, preferred_element_type=jnp.float32)
        c2 = jnp.sum(cbt * cbt, axis=0, keepdims=True)
        d2 = (l2 - s) + c2
        mt = jnp.min(d2, axis=1, keepdims=True)
        iota = lax.broadcasted_iota(jnp.int32, (_M, _KT), 1) + k * _KT
        it = jnp.min(jnp.where(d2 == mt, iota, jnp.int32(2**30)),
                     axis=1, keepdims=True)
        better = mt < m
        return jnp.where(better, mt, m), jnp.where(better, it, am)

    m0 = jnp.full((_M, 1), jnp.inf, jnp.float32)
    am0 = jnp.zeros((_M, 1), jnp.int32)
    m, am = lax.fori_loop(0, _K // _KT, tile, (m0, am0))
    idx_ref[...] = am[:, 0]
    d2_ref[...] = m[:, 0]


def _k1_call(x2d, w_in, b_in2, cbt, interpret=False):
    grid = (_N // _M,)
    return pl.pallas_call(
        _k1_body,
        grid=grid,
        in_specs=[
            pl.BlockSpec((_M, _D), lambda i: (i, 0)),
            pl.BlockSpec((_D, _CD), lambda i: (0, 0)),
            pl.BlockSpec((1, _CD), lambda i: (0, 0)),
            pl.BlockSpec((_CD, _K), lambda i: (0, 0)),
        ],
        out_specs=[
            pl.BlockSpec((_M,), lambda i: (i,)),
            pl.BlockSpec((_M,), lambda i: (i,)),
        ],
        out_shape=[
            jax.ShapeDtypeStruct((_N,), jnp.int32),
            jax.ShapeDtypeStruct((_N,), jnp.float32),
        ],
        interpret=interpret,
    )(x2d, w_in, b_in2, cbt)


_NW = 32
_BPW = _N // _NW
_CHUNK = 96


def _sc_gather(codebook, idx):
    mesh = plsc.VectorSubcoreMesh(core_axis_name="c", subcore_axis_name="s")

    @functools.partial(
        pl.kernel,
        mesh=mesh,
        out_type=jax.ShapeDtypeStruct((_N, _CD), jnp.float32),
        scratch_types=[
            pltpu.VMEM((_BPW,), jnp.int32),
            pltpu.VMEM((_BPW, _CD), jnp.float32),
            pltpu.SemaphoreType.DMA,
        ],
    )
    def gather_k(table_hbm, idx_hbm, out_hbm, idx_v, rows_v, sem):
        wid = lax.axis_index("s") * 2 + lax.axis_index("c")
        base = wid * _BPW
        pltpu.sync_copy(idx_hbm.at[pl.ds(base, _BPW)], idx_v)
        copies = []
        for c in range(_BPW // _CHUNK):
            copies.append(pltpu.async_copy(
                table_hbm.at[idx_v.at[pl.ds(c * _CHUNK, _CHUNK)]],
                rows_v.at[pl.ds(c * _CHUNK, _CHUNK), :],
                sem))
        for cp in copies:
            cp.wait()
        pltpu.sync_copy(rows_v, out_hbm.at[pl.ds(base, _BPW)])

    return gather_k(codebook, idx)


def _k3_body(q_ref, wout_ref, bout_ref, d2_ref, qf_ref, loss_ref):
    i = pl.program_id(0)
    qf_ref[...] = jnp.dot(q_ref[...], wout_ref[...],
                          preferred_element_type=jnp.float32) + bout_ref[...]
    part = jnp.sum(d2_ref[...]).reshape(1, 1)

    @pl.when(i == 0)
    def _():
        loss_ref[...] = jnp.zeros((1, 1), jnp.float32)

    acc = loss_ref[...] + part

    @pl.when(i < _N // _M - 1)
    def _():
        loss_ref[...] = acc

    @pl.when(i == _N // _M - 1)
    def _():
        loss_ref[...] = acc * (1.25 / (_N * _CD))


def _k3_call(quant, w_out, b_out2, d2min, interpret=False):
    grid = (_N // _M,)
    return pl.pallas_call(
        _k3_body,
        grid=grid,
        in_specs=[
            pl.BlockSpec((_M, _CD), lambda i: (i, 0)),
            pl.BlockSpec((_CD, _D), lambda i: (0, 0)),
            pl.BlockSpec((1, _D), lambda i: (0, 0)),
            pl.BlockSpec((_M,), lambda i: (i,)),
        ],
        out_specs=[
            pl.BlockSpec((_M, _D), lambda i: (i, 0)),
            pl.BlockSpec((1, 1), lambda i: (0, 0)),
        ],
        out_shape=[
            jax.ShapeDtypeStruct((_N, _D), jnp.int32),
            jax.ShapeDtypeStruct((1, 1), jnp.float32),
        ],
        interpret=interpret,
    )(quant, w_out, b_out2, d2min)


def kernel(x, W_in, b_in, W_out, b_out, codebook):
    x2d = x.reshape(_N, _D)
    idx, d2min = _k1_call(x2d, W_in, b_in.reshape(1, _CD), codebook.T)
    quant = _sc_gather(codebook, idx)
    qf2d, loss = _k3_call(quant, W_out, b_out.reshape(1, _D), d2min)
    return qf2d.reshape(_B, _T, _D), idx.reshape(_B, _T), loss.reshape(())


# speedup vs baseline: 1.0497x; 1.0497x over previous
import functools

import jax
import jax.numpy as jnp
from jax import lax
from jax.experimental import pallas as pl
from jax.experimental.pallas import tpu as pltpu
from jax.experimental.pallas import tpu_sc as plsc

_B, _T, _D, _CD, _K = 16, 576, 768, 64, 8192
_N = _B * _T

_M = 512
_KT = 2048


def _k1_body(x_ref, win_ref, bin_ref, cbt_ref, idx_ref, d2_ref):
    lat = jnp.dot(x_ref[...], win_ref[...],
                  preferred_element_type=jnp.float32) + bin_ref[...]
    l2 = jnp.sum(lat * lat, axis=1, keepdims=True)
    lat2 = lat * 2.0

    def tile(k, carry):
        m, am = carry
        cbt = cbt_ref[:, pl.ds(k * _KT, _KT)]
        s = jnp.dot(lat2, cbt, preferred_element_type=jnp.float32)
        c2 = jnp.sum(cbt * cbt, axis=0, keepdims=True)
        d2 = (l2 - s) + c2
        mt = jnp.min(d2, axis=1, keepdims=True)
        iota = lax.broadcasted_iota(jnp.int32, (_M, _KT), 1) + k * _KT
        it = jnp.min(jnp.where(d2 == mt, iota, jnp.int32(2**30)),
                     axis=1, keepdims=True)
        better = mt < m
        return jnp.where(better, mt, m), jnp.where(better, it, am)

    m0 = jnp.full((_M, 1), jnp.inf, jnp.float32)
    am0 = jnp.zeros((_M, 1), jnp.int32)
    m, am = lax.fori_loop(0, _K // _KT, tile, (m0, am0))
    idx_ref[...] = am[:, 0]
    d2_ref[...] = m[:, 0]


def _k1_call(x2d, w_in, b_in2, cbt, interpret=False):
    grid = (_N // _M,)
    return pl.pallas_call(
        _k1_body,
        grid=grid,
        in_specs=[
            pl.BlockSpec((_M, _D), lambda i: (i, 0)),
            pl.BlockSpec((_D, _CD), lambda i: (0, 0)),
            pl.BlockSpec((1, _CD), lambda i: (0, 0)),
            pl.BlockSpec((_CD, _K), lambda i: (0, 0)),
        ],
        out_specs=[
            pl.BlockSpec((_M,), lambda i: (i,)),
            pl.BlockSpec((_M,), lambda i: (i,)),
        ],
        out_shape=[
            jax.ShapeDtypeStruct((_N,), jnp.int32),
            jax.ShapeDtypeStruct((_N,), jnp.float32),
        ],
        interpret=interpret,
    )(x2d, w_in, b_in2, cbt)


_NW = 32
_BPW = _N // _NW
_CHUNK = 96


def _sc_gather(codebook_packed, idx):
    mesh = plsc.VectorSubcoreMesh(core_axis_name="c", subcore_axis_name="s")

    @functools.partial(
        pl.kernel,
        mesh=mesh,
        out_type=jax.ShapeDtypeStruct((_N, 2 * _CD), jnp.float32),
        scratch_types=[
            pltpu.VMEM((_BPW,), jnp.int32),
            pltpu.VMEM((_BPW,), jnp.int32),
            pltpu.VMEM((_BPW, 2 * _CD), jnp.float32),
            pltpu.SemaphoreType.DMA,
        ],
    )
    def gather_k(table_hbm, idx_hbm, out_hbm, idx_v, pidx_v, rows_v, sem):
        wid = lax.axis_index("s") * 2 + lax.axis_index("c")
        base = wid * _BPW
        pltpu.sync_copy(idx_hbm.at[pl.ds(base, _BPW)], idx_v)
        for c in range(_BPW // 16):
            pidx_v[pl.ds(c * 16, 16)] = jnp.right_shift(
                idx_v[pl.ds(c * 16, 16)], 1)
        copies = []
        for c in range(_BPW // _CHUNK):
            copies.append(pltpu.async_copy(
                table_hbm.at[pidx_v.at[pl.ds(c * _CHUNK, _CHUNK)]],
                rows_v.at[pl.ds(c * _CHUNK, _CHUNK), :],
                sem))
        for cp in copies:
            cp.wait()
        pltpu.sync_copy(rows_v, out_hbm.at[pl.ds(base, _BPW)])

    return gather_k(codebook_packed, idx)


def _k3_body(pq_ref, idx_ref, wout_ref, bout_ref, d2_ref, qf_ref, loss_ref):
    i = pl.program_id(0)
    odd = (idx_ref[...] & 1)[:, None] == 1
    pq = pq_ref[...]
    q = jnp.where(odd, pq[:, _CD:], pq[:, :_CD])
    qf_ref[...] = jnp.dot(q, wout_ref[...],
                          preferred_element_type=jnp.float32) + bout_ref[...]
    part = jnp.sum(d2_ref[...]).reshape(1, 1)

    @pl.when(i == 0)
    def _():
        loss_ref[...] = jnp.zeros((1, 1), jnp.float32)

    acc = loss_ref[...] + part

    @pl.when(i < _N // _M - 1)
    def _():
        loss_ref[...] = acc

    @pl.when(i == _N // _M - 1)
    def _():
        loss_ref[...] = acc * (1.25 / (_N * _CD))


def _k3_call(pquant, idx, w_out, b_out2, d2min, interpret=False):
    grid = (_N // _M,)
    return pl.pallas_call(
        _k3_body,
        grid=grid,
        in_specs=[
            pl.BlockSpec((_M, 2 * _CD), lambda i: (i, 0)),
            pl.BlockSpec((_M,), lambda i: (i,)),
            pl.BlockSpec((_CD, _D), lambda i: (0, 0)),
            pl.BlockSpec((1, _D), lambda i: (0, 0)),
            pl.BlockSpec((_M,), lambda i: (i,)),
        ],
        out_specs=[
            pl.BlockSpec((_M, _D), lambda i: (i, 0)),
            pl.BlockSpec((1, 1), lambda i: (0, 0)),
        ],
        out_shape=[
            jax.ShapeDtypeStruct((_N, _D), jnp.float32),
            jax.ShapeDtypeStruct((1, 1), jnp.float32),
        ],
        interpret=interpret,
    )(pquant, idx, w_out, b_out2, d2min)


def kernel(x, W_in, b_in, W_out, b_out, codebook):
    x2d = x.reshape(_N, _D)
    idx, d2min = _k1_call(x2d, W_in, b_in.reshape(1, _CD), codebook.T)
    pquant = _sc_gather(codebook.reshape(_K // 2, 2 * _CD), idx)
    qf2d, loss = _k3_call(pquant, idx, W_out, b_out.reshape(1, _D), d2min)
    return qf2d.reshape(_B, _T, _D), idx.reshape(_B, _T), loss.reshape(())


# speedup vs baseline: 1.1402x; 1.0862x over previous
import functools

import jax
import jax.numpy as jnp
from jax import lax
from jax.experimental import pallas as pl
from jax.experimental.pallas import tpu as pltpu
from jax.experimental.pallas import tpu_sc as plsc

_B, _T, _D, _CD, _K = 16, 576, 768, 64, 8192
_N = _B * _T

_M = 512
_KT = 2048


def _k1_body(x_ref, win_ref, bin_ref, cbt_ref, idx_ref, d2_ref,
             acc_ref, tid_ref):
    lat = jnp.dot(x_ref[...], win_ref[...],
                  preferred_element_type=jnp.float32) + bin_ref[...]
    l2 = jnp.sum(lat * lat, axis=1, keepdims=True)
    lat2 = lat * 2.0

    def dist(k):
        cbt = cbt_ref[:, pl.ds(k * _KT, _KT)]
        s = jnp.dot(lat2, cbt, preferred_element_type=jnp.float32)
        c2 = jnp.sum(cbt * cbt, axis=0, keepdims=True)
        return (l2 - s) + c2

    acc_ref[...] = dist(0)
    tid_ref[...] = jnp.zeros((_M, _KT), jnp.int32)

    def tile(k, carry):
        d2 = dist(k)
        av = acc_ref[...]
        lt = d2 < av
        acc_ref[...] = jnp.minimum(av, d2)
        tid_ref[...] = jnp.where(lt, k, tid_ref[...])
        return carry

    lax.fori_loop(1, _K // _KT, tile, 0)

    av = acc_ref[...]
    m = jnp.min(av, axis=1, keepdims=True)
    gi = lax.broadcasted_iota(jnp.int32, (_M, _KT), 1) + tid_ref[...] * _KT
    it = jnp.min(jnp.where(av == m, gi, jnp.int32(2**30)),
                 axis=1, keepdims=True)
    idx_ref[...] = it[:, 0]
    d2_ref[...] = m[:, 0]


def _k1_call(x2d, w_in, b_in2, cbt, interpret=False):
    grid = (_N // _M,)
    return pl.pallas_call(
        _k1_body,
        grid=grid,
        in_specs=[
            pl.BlockSpec((_M, _D), lambda i: (i, 0)),
            pl.BlockSpec((_D, _CD), lambda i: (0, 0)),
            pl.BlockSpec((1, _CD), lambda i: (0, 0)),
            pl.BlockSpec((_CD, _K), lambda i: (0, 0)),
        ],
        out_specs=[
            pl.BlockSpec((_M,), lambda i: (i,)),
            pl.BlockSpec((_M,), lambda i: (i,)),
        ],
        out_shape=[
            jax.ShapeDtypeStruct((_N,), jnp.int32),
            jax.ShapeDtypeStruct((_N,), jnp.float32),
        ],
        scratch_shapes=[
            pltpu.VMEM((_M, _KT), jnp.float32),
            pltpu.VMEM((_M, _KT), jnp.int32),
        ],
        interpret=interpret,
    )(x2d, w_in, b_in2, cbt)


_NW = 32
_BPW = _N // _NW
_CHUNK = 96


def _sc_gather(codebook_packed, idx):
    mesh = plsc.VectorSubcoreMesh(core_axis_name="c", subcore_axis_name="s")

    @functools.partial(
        pl.kernel,
        mesh=mesh,
        out_type=jax.ShapeDtypeStruct((_N, 2 * _CD), jnp.float32),
        scratch_types=[
            pltpu.VMEM((_BPW,), jnp.int32),
            pltpu.VMEM((_BPW,), jnp.int32),
            pltpu.VMEM((_BPW, 2 * _CD), jnp.float32),
            pltpu.SemaphoreType.DMA,
        ],
    )
    def gather_k(table_hbm, idx_hbm, out_hbm, idx_v, pidx_v, rows_v, sem):
        wid = lax.axis_index("s") * 2 + lax.axis_index("c")
        base = wid * _BPW
        pltpu.sync_copy(idx_hbm.at[pl.ds(base, _BPW)], idx_v)
        for c in range(_BPW // 16):
            pidx_v[pl.ds(c * 16, 16)] = jnp.right_shift(
                idx_v[pl.ds(c * 16, 16)], 1)
        copies = []
        for c in range(_BPW // _CHUNK):
            copies.append(pltpu.async_copy(
                table_hbm.at[pidx_v.at[pl.ds(c * _CHUNK, _CHUNK)]],
                rows_v.at[pl.ds(c * _CHUNK, _CHUNK), :],
                sem))
        for cp in copies:
            cp.wait()
        pltpu.sync_copy(rows_v, out_hbm.at[pl.ds(base, _BPW)])

    return gather_k(codebook_packed, idx)


def _k3_body(pq_ref, idx_ref, wout_ref, bout_ref, d2_ref, qf_ref, loss_ref):
    i = pl.program_id(0)
    odd = (idx_ref[...] & 1)[:, None] == 1
    pq = pq_ref[...]
    q = jnp.where(odd, pq[:, _CD:], pq[:, :_CD])
    qf_ref[...] = jnp.dot(q, wout_ref[...],
                          preferred_element_type=jnp.float32) + bout_ref[...]
    part = jnp.sum(d2_ref[...]).reshape(1, 1)

    @pl.when(i == 0)
    def _():
        loss_ref[...] = jnp.zeros((1, 1), jnp.float32)

    acc = loss_ref[...] + part

    @pl.when(i < _N // _M - 1)
    def _():
        loss_ref[...] = acc

    @pl.when(i == _N // _M - 1)
    def _():
        loss_ref[...] = acc * (1.25 / (_N * _CD))


def _k3_call(pquant, idx, w_out, b_out2, d2min, interpret=False):
    grid = (_N // _M,)
    return pl.pallas_call(
        _k3_body,
        grid=grid,
        in_specs=[
            pl.BlockSpec((_M, 2 * _CD), lambda i: (i, 0)),
            pl.BlockSpec((_M,), lambda i: (i,)),
            pl.BlockSpec((_CD, _D), lambda i: (0, 0)),
            pl.BlockSpec((1, _D), lambda i: (0, 0)),
            pl.BlockSpec((_M,), lambda i: (i,)),
        ],
        out_specs=[
            pl.BlockSpec((_M, _D), lambda i: (i, 0)),
            pl.BlockSpec((1, 1), lambda i: (0, 0)),
        ],
        out_shape=[
            jax.ShapeDtypeStruct((_N, _D), jnp.float32),
            jax.ShapeDtypeStruct((1, 1), jnp.float32),
        ],
        interpret=interpret,
    )(pquant, idx, w_out, b_out2, d2min)


def kernel(x, W_in, b_in, W_out, b_out, codebook):
    x2d = x.reshape(_N, _D)
    idx, d2min = _k1_call(x2d, W_in, b_in.reshape(1, _CD), codebook.T)
    pquant = _sc_gather(codebook.reshape(_K // 2, 2 * _CD), idx)
    qf2d, loss = _k3_call(pquant, idx, W_out, b_out.reshape(1, _D), d2min)
    return qf2d.reshape(_B, _T, _D), idx.reshape(_B, _T), loss.reshape(())


# speedup vs baseline: 1.2085x; 1.0599x over previous
import functools

import jax
import jax.numpy as jnp
from jax import lax
from jax.experimental import pallas as pl
from jax.experimental.pallas import tpu as pltpu
from jax.experimental.pallas import tpu_sc as plsc

_B, _T, _D, _CD, _K = 16, 576, 768, 64, 8192
_N = _B * _T

_M = 512
_M3 = 1024
_KT = 2048


def _k1_body(x_ref, win_ref, bin_ref, cbt_ref, idx_ref, d2_ref,
             acc_ref, tid_ref, c2_ref):
    @pl.when(pl.program_id(0) == 0)
    def _():
        cb = cbt_ref[...]
        c2_ref[...] = jnp.sum(cb * cb, axis=0, keepdims=True)

    lat = jnp.dot(x_ref[...], win_ref[...],
                  preferred_element_type=jnp.float32) + bin_ref[...]
    l2 = jnp.sum(lat * lat, axis=1, keepdims=True)
    lat2 = lat * 2.0

    def dist(k):
        cbt = cbt_ref[:, pl.ds(k * _KT, _KT)]
        s = jnp.dot(lat2, cbt, preferred_element_type=jnp.float32)
        c2 = c2_ref[:, pl.ds(k * _KT, _KT)]
        return (l2 - s) + c2

    acc_ref[...] = dist(0)
    tid_ref[...] = jnp.zeros((_M, _KT), jnp.float32)

    def tile(k, carry):
        d2 = dist(k)
        av = acc_ref[...]
        lt = d2 < av
        acc_ref[...] = jnp.minimum(av, d2)
        tid_ref[...] = jnp.where(lt, k.astype(jnp.float32), tid_ref[...])
        return carry

    lax.fori_loop(1, _K // _KT, tile, 0)

    av = acc_ref[...]
    m = jnp.min(av, axis=1, keepdims=True)
    iota_f = lax.broadcasted_iota(jnp.int32, (_M, _KT), 1).astype(jnp.float32)
    gi = iota_f + tid_ref[...] * float(_KT)
    it = jnp.min(jnp.where(av == m, gi, jnp.float32(2.0**30)),
                 axis=1, keepdims=True)
    idx_ref[...] = it[:, 0].astype(jnp.int32)
    d2_ref[...] = m[:, 0]


def _k1_call(x2d, w_in, b_in2, cbt, interpret=False):
    grid = (_N // _M,)
    return pl.pallas_call(
        _k1_body,
        grid=grid,
        in_specs=[
            pl.BlockSpec((_M, _D), lambda i: (i, 0)),
            pl.BlockSpec((_D, _CD), lambda i: (0, 0)),
            pl.BlockSpec((1, _CD), lambda i: (0, 0)),
            pl.BlockSpec((_CD, _K), lambda i: (0, 0)),
        ],
        out_specs=[
            pl.BlockSpec((_M,), lambda i: (i,)),
            pl.BlockSpec((_M,), lambda i: (i,)),
        ],
        out_shape=[
            jax.ShapeDtypeStruct((_N,), jnp.int32),
            jax.ShapeDtypeStruct((_N,), jnp.float32),
        ],
        scratch_shapes=[
            pltpu.VMEM((_M, _KT), jnp.float32),
            pltpu.VMEM((_M, _KT), jnp.float32),
            pltpu.VMEM((1, _K), jnp.float32),
        ],
        interpret=interpret,
    )(x2d, w_in, b_in2, cbt)


_NW = 32
_BPW = _N // _NW
_CHUNK = 96


def _sc_gather(codebook_packed, idx):
    mesh = plsc.VectorSubcoreMesh(core_axis_name="c", subcore_axis_name="s")

    @functools.partial(
        pl.kernel,
        mesh=mesh,
        out_type=jax.ShapeDtypeStruct((_N, 2 * _CD), jnp.float32),
        scratch_types=[
            pltpu.VMEM((_BPW,), jnp.int32),
            pltpu.VMEM((_BPW,), jnp.int32),
            pltpu.VMEM((_BPW, 2 * _CD), jnp.float32),
            pltpu.SemaphoreType.DMA,
        ],
    )
    def gather_k(table_hbm, idx_hbm, out_hbm, idx_v, pidx_v, rows_v, sem):
        wid = lax.axis_index("s") * 2 + lax.axis_index("c")
        base = wid * _BPW
        pltpu.sync_copy(idx_hbm.at[pl.ds(base, _BPW)], idx_v)
        for c in range(_BPW // 16):
            pidx_v[pl.ds(c * 16, 16)] = jnp.right_shift(
                idx_v[pl.ds(c * 16, 16)], 1)
        copies = []
        for c in range(_BPW // _CHUNK):
            copies.append(pltpu.async_copy(
                table_hbm.at[pidx_v.at[pl.ds(c * _CHUNK, _CHUNK)]],
                rows_v.at[pl.ds(c * _CHUNK, _CHUNK), :],
                sem))
        for cp in copies:
            cp.wait()
        pltpu.sync_copy(rows_v, out_hbm.at[pl.ds(base, _BPW)])

    return gather_k(codebook_packed, idx)


def _k3_body(pq_ref, idx_ref, wout_ref, bout_ref, d2_ref, qf_ref, loss_ref):
    i = pl.program_id(0)
    odd = (idx_ref[...] & 1)[:, None] == 1
    pq = pq_ref[...]
    q = jnp.where(odd, pq[:, _CD:], pq[:, :_CD])
    qf_ref[...] = jnp.dot(q, wout_ref[...],
                          preferred_element_type=jnp.float32) + bout_ref[...]
    part = jnp.sum(d2_ref[...]).reshape(1, 1)

    @pl.when(i == 0)
    def _():
        loss_ref[...] = jnp.zeros((1, 1), jnp.float32)

    acc = loss_ref[...] + part

    @pl.when(i < _N // _M3 - 1)
    def _():
        loss_ref[...] = acc

    @pl.when(i == _N // _M3 - 1)
    def _():
        loss_ref[...] = acc * (1.25 / (_N * _CD))


def _k3_call(pquant, idx, w_out, b_out2, d2min, interpret=False):
    grid = (_N // _M3,)
    return pl.pallas_call(
        _k3_body,
        grid=grid,
        in_specs=[
            pl.BlockSpec((_M3, 2 * _CD), lambda i: (i, 0)),
            pl.BlockSpec((_M3,), lambda i: (i,)),
            pl.BlockSpec((_CD, _D), lambda i: (0, 0)),
            pl.BlockSpec((1, _D), lambda i: (0, 0)),
            pl.BlockSpec((_M3,), lambda i: (i,)),
        ],
        out_specs=[
            pl.BlockSpec((_M3, _D), lambda i: (i, 0)),
            pl.BlockSpec((1, 1), lambda i: (0, 0)),
        ],
        out_shape=[
            jax.ShapeDtypeStruct((_N, _D), jnp.float32),
            jax.ShapeDtypeStruct((1, 1), jnp.float32),
        ],
        interpret=interpret,
    )(pquant, idx, w_out, b_out2, d2min)


def kernel(x, W_in, b_in, W_out, b_out, codebook):
    x2d = x.reshape(_N, _D)
    idx, d2min = _k1_call(x2d, W_in, b_in.reshape(1, _CD), codebook.T)
    pquant = _sc_gather(codebook.reshape(_K // 2, 2 * _CD), idx)
    qf2d, loss = _k3_call(pquant, idx, W_out, b_out.reshape(1, _D), d2min)
    return qf2d.reshape(_B, _T, _D), idx.reshape(_B, _T), loss.reshape(())


# speedup vs baseline: 1.5409x; 1.2751x over previous
import functools

import jax
import jax.numpy as jnp
from jax import lax
from jax.experimental import pallas as pl
from jax.experimental.pallas import tpu as pltpu
from jax.experimental.pallas import tpu_sc as plsc

_B, _T, _D, _CD, _K = 16, 576, 768, 64, 8192
_N = _B * _T

_M = 1024
_M3 = 1024


def _k1_body(x_ref, win_ref, bin_ref, cb_ref, idx_ref, lat_ref, c2_ref):
    @pl.when(pl.program_id(0) == 0)
    def _():
        cbv = cb_ref[...]
        c2_ref[...] = jnp.sum(cbv * cbv, axis=0, keepdims=True)

    lat = jnp.dot(x_ref[...], win_ref[...],
                  preferred_element_type=jnp.float32) + bin_ref[...]
    l2 = jnp.sum(lat * lat, axis=1, keepdims=True)
    lat2 = lat * 2.0

    cb = cb_ref[...]
    c2 = c2_ref[...]
    h = _M // 2
    s_a = jnp.dot(lat2[:h], cb, preferred_element_type=jnp.float32)
    s_b = jnp.dot(lat2[h:], cb, preferred_element_type=jnp.float32)
    d2_a = (l2[:h] - s_a) + c2
    d2_b = (l2[h:] - s_b) + c2
    idx_ref[pl.ds(0, h)] = jnp.argmin(d2_a, axis=1).astype(jnp.int32)
    idx_ref[pl.ds(h, h)] = jnp.argmin(d2_b, axis=1).astype(jnp.int32)
    lat_ref[...] = lat


def _k1_call(x2d, w_in, b_in2, cbt, interpret=False):
    grid = (_N // _M,)
    return pl.pallas_call(
        _k1_body,
        grid=grid,
        in_specs=[
            pl.BlockSpec((_M, _D), lambda i: (i, 0)),
            pl.BlockSpec((_D, _CD), lambda i: (0, 0)),
            pl.BlockSpec((1, _CD), lambda i: (0, 0)),
            pl.BlockSpec((_CD, _K), lambda i: (0, 0)),
        ],
        out_specs=[
            pl.BlockSpec((_M,), lambda i: (i,)),
            pl.BlockSpec((_M, _CD), lambda i: (i, 0)),
        ],
        out_shape=[
            jax.ShapeDtypeStruct((_N,), jnp.int32),
            jax.ShapeDtypeStruct((_N, _CD), jnp.float32),
        ],
        scratch_shapes=[
            pltpu.VMEM((1, _K), jnp.float32),
        ],
        interpret=interpret,
    )(x2d, w_in, b_in2, cbt)


_NW = 32
_BPW = _N // _NW
_CHUNK = 96


def _sc_gather(codebook_packed, idx):
    mesh = plsc.VectorSubcoreMesh(core_axis_name="c", subcore_axis_name="s")

    @functools.partial(
        pl.kernel,
        mesh=mesh,
        out_type=jax.ShapeDtypeStruct((_N, 2 * _CD), jnp.float32),
        scratch_types=[
            pltpu.VMEM((_BPW,), jnp.int32),
            pltpu.VMEM((_BPW,), jnp.int32),
            pltpu.VMEM((_BPW, 2 * _CD), jnp.float32),
            pltpu.SemaphoreType.DMA,
        ],
    )
    def gather_k(table_hbm, idx_hbm, out_hbm, idx_v, pidx_v, rows_v, sem):
        wid = lax.axis_index("s") * 2 + lax.axis_index("c")
        base = wid * _BPW
        pltpu.sync_copy(idx_hbm.at[pl.ds(base, _BPW)], idx_v)
        for c in range(_BPW // 16):
            pidx_v[pl.ds(c * 16, 16)] = jnp.right_shift(
                idx_v[pl.ds(c * 16, 16)], 1)
        copies = []
        for c in range(_BPW // _CHUNK):
            copies.append(pltpu.async_copy(
                table_hbm.at[pidx_v.at[pl.ds(c * _CHUNK, _CHUNK)]],
                rows_v.at[pl.ds(c * _CHUNK, _CHUNK), :],
                sem))
        for cp in copies:
            cp.wait()
        pltpu.sync_copy(rows_v, out_hbm.at[pl.ds(base, _BPW)])

    return gather_k(codebook_packed, idx)


def _k3_body(pq_ref, idx_ref, wout_ref, bout_ref, lat_ref, qf_ref, loss_ref):
    i = pl.program_id(0)
    odd = (idx_ref[...] & 1)[:, None] == 1
    pq = pq_ref[...]
    q = jnp.where(odd, pq[:, _CD:], pq[:, :_CD])
    qf_ref[...] = jnp.dot(q, wout_ref[...],
                          preferred_element_type=jnp.float32) + bout_ref[...]
    e = q - lat_ref[...]
    part = jnp.sum(e * e).reshape(1, 1)

    @pl.when(i == 0)
    def _():
        loss_ref[...] = jnp.zeros((1, 1), jnp.float32)

    acc = loss_ref[...] + part

    @pl.when(i < _N // _M3 - 1)
    def _():
        loss_ref[...] = acc

    @pl.when(i == _N // _M3 - 1)
    def _():
        loss_ref[...] = acc * (1.25 / (_N * _CD))


def _k3_call(pquant, idx, w_out, b_out2, lat, interpret=False):
    grid = (_N // _M3,)
    return pl.pallas_call(
        _k3_body,
        grid=grid,
        in_specs=[
            pl.BlockSpec((_M3, 2 * _CD), lambda i: (i, 0)),
            pl.BlockSpec((_M3,), lambda i: (i,)),
            pl.BlockSpec((_CD, _D), lambda i: (0, 0)),
            pl.BlockSpec((1, _D), lambda i: (0, 0)),
            pl.BlockSpec((_M3, _CD), lambda i: (i, 0)),
        ],
        out_specs=[
            pl.BlockSpec((_M3, _D), lambda i: (i, 0)),
            pl.BlockSpec((1, 1), lambda i: (0, 0)),
        ],
        out_shape=[
            jax.ShapeDtypeStruct((_N, _D), jnp.float32),
            jax.ShapeDtypeStruct((1, 1), jnp.float32),
        ],
        interpret=interpret,
    )(pquant, idx, w_out, b_out2, lat)


def kernel(x, W_in, b_in, W_out, b_out, codebook):
    x2d = x.reshape(_N, _D)
    idx, lat = _k1_call(x2d, W_in, b_in.reshape(1, _CD), codebook.T)
    pquant = _sc_gather(codebook.reshape(_K // 2, 2 * _CD), idx)
    qf2d, loss = _k3_call(pquant, idx, W_out, b_out.reshape(1, _D), lat)
    return qf2d.reshape(_B, _T, _D), idx.reshape(_B, _T), loss.reshape(())
